# Initial kernel scaffold; baseline (speedup 1.0000x reference)
import jax
import jax.numpy as jnp
from jax.experimental import pallas as pl


def kernel(x, edge_index, neg_edge_index, W1_self, W1_neigh, b1, W2_self, W2_neigh, b2):
    raise NotImplementedError("write your pallas kernel here")


# speedup vs baseline: 45.6107x; 45.6107x over previous
import functools

import jax
import jax.numpy as jnp
from jax import lax
from jax.experimental import pallas as pl
from jax.experimental.pallas import tpu as pltpu
from jax.experimental.pallas import tpu_sc as plsc

N = 100000
E = 3200000
NC, NS = 2, 16
NW = NC * NS
W = 100
RCHUNK = 40
C = RCHUNK * W
ROWS_PW = E // W // NW
ITERS = ROWS_PW // RCHUNK
EPW = E // NW
RPT = 6272
N_PAD = NS * RPT

@functools.lru_cache(maxsize=None)
def _mesh():
    return plsc.VectorSubcoreMesh(
        core_axis_name="c", subcore_axis_name="s", num_cores=NC,
        num_subcores=NS)


@functools.lru_cache(maxsize=None)
def _make_scatter(D):
    @functools.partial(
        pl.kernel,
        out_type=jax.ShapeDtypeStruct((NC, N_PAD, D), jnp.float32),
        mesh=_mesh(),
        compiler_params=pltpu.CompilerParams(use_tc_tiling_on_sc=False, needs_layout_passes=False),
        scratch_types=[
            pltpu.VMEM((RCHUNK, W), jnp.int32),
            pltpu.VMEM((RCHUNK, W), jnp.int32),
            pltpu.VMEM((C, D), jnp.float32),
            pltpu.VMEM_SHARED((N_PAD, D), jnp.float32),
            pltpu.SemaphoreType.DMA,
            pltpu.SemaphoreType.DMA,
        ],
    )
    def sk(table, src, dst, zrows, out, src_v, dst_v, rows_v, acc, gsem, ssem):
        cid = lax.axis_index("c")
        sid = lax.axis_index("s")
        wid = sid * NC + cid
        r0 = sid * RPT
        pltpu.sync_copy(zrows, acc.at[pl.ds(r0, RPT)])
        plsc.subcore_barrier()
        rbase = wid * ROWS_PW

        def body(i, carry):
            roff = rbase + i * RCHUNK
            pltpu.sync_copy(src.at[pl.ds(roff, RCHUNK)], src_v)
            pltpu.sync_copy(dst.at[pl.ds(roff, RCHUNK)], dst_v)
            g = [
                pltpu.async_copy(
                    table.at[src_v.at[j]], rows_v.at[pl.ds(j * W, W)], gsem)
                for j in range(RCHUNK)
            ]
            for d in g:
                d.wait()
            s = [
                pltpu.async_copy(
                    rows_v.at[pl.ds(j * W, W)], acc.at[dst_v.at[j]], ssem,
                    add=True)
                for j in range(RCHUNK)
            ]
            for d in s:
                d.wait()
            return carry

        lax.fori_loop(0, ITERS, body, 0)
        plsc.subcore_barrier()
        pltpu.sync_copy(acc.at[pl.ds(r0, RPT)], out.at[cid, pl.ds(r0, RPT)])

    return sk


@functools.lru_cache(maxsize=None)
def _make_score():
    @functools.partial(
        pl.kernel,
        out_type=jax.ShapeDtypeStruct((E,), jnp.float32),
        mesh=_mesh(),
        compiler_params=pltpu.CompilerParams(use_tc_tiling_on_sc=False, needs_layout_passes=False),
        scratch_types=[
            pltpu.VMEM((RCHUNK, W), jnp.int32),
            pltpu.VMEM((RCHUNK, W), jnp.int32),
            pltpu.VMEM((C, 8), jnp.float32),
            pltpu.VMEM((C, 8), jnp.float32),
            pltpu.VMEM((C,), jnp.float32),
            pltpu.SemaphoreType.DMA,
            pltpu.SemaphoreType.DMA,
        ],
    )
    def _score_k(h2, src, dst, out, src_v, dst_v, hu_v, hv_v, sc_v, s1, s2):
        cid = lax.axis_index("c")
        sid = lax.axis_index("s")
        wid = sid * NC + cid
        rbase = wid * ROWS_PW
        iot = lax.iota(jnp.int32, 16)

        def body(i, carry):
            roff = rbase + i * RCHUNK
            pltpu.sync_copy(src.at[pl.ds(roff, RCHUNK)], src_v)
            pltpu.sync_copy(dst.at[pl.ds(roff, RCHUNK)], dst_v)
            g1 = [
                pltpu.async_copy(h2.at[src_v.at[j]], hu_v.at[pl.ds(j * W, W)], s1)
                for j in range(RCHUNK)
            ]
            g2 = [
                pltpu.async_copy(h2.at[dst_v.at[j]], hv_v.at[pl.ds(j * W, W)], s2)
                for j in range(RCHUNK)
            ]
            for d in g1 + g2:
                d.wait()

            def dot_body(t, carry2):
                rows16 = t * 16 + iot
                s = jnp.zeros((16,), jnp.float32)
                for j in range(8):
                    cj = jnp.full((16,), j, jnp.int32)
                    s = s + (plsc.load_gather(hu_v, [rows16, cj])
                             * plsc.load_gather(hv_v, [rows16, cj]))
                sc_v[pl.ds(t * 16, 16)] = s
                return carry2

            lax.fori_loop(0, C // 16, dot_body, 0)
            pltpu.sync_copy(sc_v, out.at[pl.ds(wid * EPW + i * C, C)])
            return carry

        lax.fori_loop(0, ITERS, body, 0)

    return _score_k


BN = 2000


def _combine_body(acc_ref, y1_ref, deg_ref):
    a = acc_ref[0] + acc_ref[1]
    deg = a[:, 4:5]
    y1_ref[...] = a[:, 0:4] / jnp.maximum(deg, 1.0)
    deg_ref[...] = deg


_combine = pl.pallas_call(
    _combine_body,
    grid=(N // BN,),
    in_specs=[pl.BlockSpec((2, BN, 8), lambda i: (0, i, 0))],
    out_specs=[
        pl.BlockSpec((BN, 4), lambda i: (i, 0)),
        pl.BlockSpec((BN, 1), lambda i: (i, 0)),
    ],
    out_shape=[
        jax.ShapeDtypeStruct((N, 4), jnp.float32),
        jax.ShapeDtypeStruct((N, 1), jnp.float32),
    ],
)


def _dense_body(x_ref, y1_ref, deg_ref, acc2_ref, w1s, w1n, w2s, w2n, b1, b2,
                h2_ref):
    f32 = jnp.float32
    c0 = jnp.dot(w1s[...], w2s[...], preferred_element_type=f32)
    c1 = (jnp.dot(w1n[...], w2s[...], preferred_element_type=f32)
          + jnp.dot(w1s[...], w2n[...], preferred_element_type=f32))
    c2 = jnp.dot(w1n[...], w2n[...], preferred_element_type=f32)
    c3 = jnp.dot(b1[...], w2s[...], preferred_element_type=f32) + b2[...]
    c4 = jnp.dot(b1[...], w2n[...], preferred_element_type=f32)
    deg = deg_ref[...]
    y2 = (acc2_ref[0] + acc2_ref[1])[:, 0:4] / jnp.maximum(deg, 1.0)
    m = (deg > 0.0).astype(f32)
    h2_ref[...] = (jnp.dot(x_ref[...], c0, preferred_element_type=f32)
                   + jnp.dot(y1_ref[...], c1, preferred_element_type=f32)
                   + jnp.dot(y2, c2, preferred_element_type=f32)
                   + c3 + m * c4)


_dense = pl.pallas_call(
    _dense_body,
    grid=(N // BN,),
    in_specs=[
        pl.BlockSpec((BN, 4), lambda i: (i, 0)),
        pl.BlockSpec((BN, 4), lambda i: (i, 0)),
        pl.BlockSpec((BN, 1), lambda i: (i, 0)),
        pl.BlockSpec((2, BN, 8), lambda i: (0, i, 0)),
        pl.BlockSpec((4, 16), lambda i: (0, 0)),
        pl.BlockSpec((4, 16), lambda i: (0, 0)),
        pl.BlockSpec((16, 8), lambda i: (0, 0)),
        pl.BlockSpec((16, 8), lambda i: (0, 0)),
        pl.BlockSpec((1, 16), lambda i: (0, 0)),
        pl.BlockSpec((1, 8), lambda i: (0, 0)),
    ],
    out_specs=pl.BlockSpec((BN, 8), lambda i: (i, 0)),
    out_shape=jax.ShapeDtypeStruct((N, 8), jnp.float32),
)


def kernel(x, edge_index, neg_edge_index, W1_self, W1_neigh, b1, W2_self,
           W2_neigh, b2):
    del neg_edge_index
    src = edge_index[0].astype(jnp.int32).reshape(E // W, W)
    dst = edge_index[1].astype(jnp.int32).reshape(E // W, W)
    xp = jnp.concatenate(
        [x, jnp.ones((N, 1), jnp.float32), jnp.zeros((N, 3), jnp.float32)],
        axis=1)
    z8 = jnp.zeros((RPT, 8), jnp.float32)

    acc1 = _make_scatter(8)(xp, src, dst, z8)
    y1, deg = _combine(acc1)
    y1p = jnp.concatenate([y1, jnp.zeros((N, 4), jnp.float32)], axis=1)
    acc2 = _make_scatter(8)(y1p, src, dst, z8)
    h2 = _dense(x, y1, deg, acc2, W1_self, W1_neigh, W2_self, W2_neigh,
                b1.reshape(1, 16), b2.reshape(1, 8))
    score = _make_score()(h2, src, dst)
    return score.reshape(E, 1)


# speedup vs baseline: 54.6370x; 1.1979x over previous
import functools

import jax
import jax.numpy as jnp
from jax import lax
from jax.experimental import pallas as pl
from jax.experimental.pallas import tpu as pltpu
from jax.experimental.pallas import tpu_sc as plsc

N = 100000
E = 3200000
NC, NS = 2, 16
NW = NC * NS
W = 80
NTR = 25
C = NTR * W
EPW = E // NW
ITERS = EPW // C
RPT = 6272
N_PAD = NS * RPT


@functools.lru_cache(maxsize=None)
def _mesh():
    return plsc.VectorSubcoreMesh(
        core_axis_name="c", subcore_axis_name="s", num_cores=NC,
        num_subcores=NS)


def _sc_params():
    return pltpu.CompilerParams(
        use_tc_tiling_on_sc=False, needs_layout_passes=False)


@functools.lru_cache(maxsize=None)
def _make_scatter(D):
    @functools.partial(
        pl.kernel,
        out_type=jax.ShapeDtypeStruct((NC, N_PAD, D), jnp.float32),
        mesh=_mesh(),
        compiler_params=_sc_params(),
        scratch_types=[
            pltpu.VMEM((C,), jnp.int32),
            pltpu.VMEM((C,), jnp.int32),
            pltpu.VMEM((C,), jnp.int32),
            pltpu.VMEM((C,), jnp.int32),
            pltpu.VMEM((C, D), jnp.float32),
            pltpu.VMEM((C, D), jnp.float32),
            pltpu.VMEM_SHARED((N_PAD, D), jnp.float32),
            pltpu.SemaphoreType.DMA,
            pltpu.SemaphoreType.DMA,
            pltpu.SemaphoreType.DMA,
            pltpu.SemaphoreType.DMA,
        ],
    )
    def sk(table, ei, zrows, out, sv0, dv0, sv1, dv1, rv0, rv1, acc,
           gs0, gs1, ss0, ss1):
        bufs = ((sv0, dv0, rv0, gs0, ss0), (sv1, dv1, rv1, gs1, ss1))
        cid = lax.axis_index("c")
        sid = lax.axis_index("s")
        wid = sid * NC + cid
        r0 = sid * RPT
        pltpu.sync_copy(zrows, acc.at[pl.ds(r0, RPT)])
        plsc.subcore_barrier()
        ebase = wid * EPW

        def chunk_fire(k, b):
            sv, dv, rv, gs, _ = bufs[b]
            off = ebase + k * C
            pltpu.sync_copy(ei.at[0, pl.ds(off, C)], sv)
            pltpu.sync_copy(ei.at[1, pl.ds(off, C)], dv)
            for j in range(NTR):
                pltpu.async_copy(
                    table.at[sv.at[pl.ds(j * W, W)]],
                    rv.at[pl.ds(j * W, W)], gs)

        def wait_gathers(b):
            _, _, rv, gs, _ = bufs[b]
            pltpu.make_async_copy(table.at[pl.ds(0, C)], rv, gs).wait()

        def scatter_fire(b):
            _, dv, rv, _, ss = bufs[b]
            for j in range(NTR):
                pltpu.async_copy(
                    rv.at[pl.ds(j * W, W)],
                    acc.at[dv.at[pl.ds(j * W, W)]], ss, add=True)

        def wait_scatters(b):
            _, _, rv, _, ss = bufs[b]
            pltpu.make_async_copy(rv, acc.at[pl.ds(0, C)], ss).wait()

        chunk_fire(0, 0)

        def body(ii, carry):
            a = 2 * ii
            wait_gathers(0)
            scatter_fire(0)

            @pl.when(ii > 0)
            def _():
                wait_scatters(1)

            chunk_fire(a + 1, 1)
            wait_gathers(1)
            scatter_fire(1)
            wait_scatters(0)

            @pl.when(ii + 1 < ITERS // 2)
            def _():
                chunk_fire(a + 2, 0)

            return carry

        lax.fori_loop(0, ITERS // 2, body, 0)
        wait_scatters(1)
        plsc.subcore_barrier()
        pltpu.sync_copy(acc.at[pl.ds(r0, RPT)], out.at[cid, pl.ds(r0, RPT)])

    return sk


@functools.lru_cache(maxsize=None)
def _make_score():
    @functools.partial(
        pl.kernel,
        out_type=jax.ShapeDtypeStruct((E,), jnp.float32),
        mesh=_mesh(),
        compiler_params=_sc_params(),
        scratch_types=[
            pltpu.VMEM((C,), jnp.int32),
            pltpu.VMEM((C,), jnp.int32),
            pltpu.VMEM((C,), jnp.int32),
            pltpu.VMEM((C,), jnp.int32),
            pltpu.VMEM((C, 8), jnp.float32),
            pltpu.VMEM((C, 8), jnp.float32),
            pltpu.VMEM((C, 8), jnp.float32),
            pltpu.VMEM((C, 8), jnp.float32),
            pltpu.VMEM((C,), jnp.float32),
            pltpu.SemaphoreType.DMA,
            pltpu.SemaphoreType.DMA,
        ],
    )
    def _score_k(h2, ei, out, sv0, dv0, sv1, dv1, hu0, hv0, hu1, hv1, sc_v,
                 g0, g1):
        bufs = ((sv0, dv0, hu0, hv0, g0), (sv1, dv1, hu1, hv1, g1))
        cid = lax.axis_index("c")
        sid = lax.axis_index("s")
        wid = sid * NC + cid
        ebase = wid * EPW
        iot = lax.iota(jnp.int32, 16)

        def chunk_fire(k, b):
            sv, dv, hu, hv, gs = bufs[b]
            off = ebase + k * C
            pltpu.sync_copy(ei.at[0, pl.ds(off, C)], sv)
            pltpu.sync_copy(ei.at[1, pl.ds(off, C)], dv)
            for j in range(NTR):
                pltpu.async_copy(
                    h2.at[sv.at[pl.ds(j * W, W)]],
                    hu.at[pl.ds(j * W, W)], gs)
                pltpu.async_copy(
                    h2.at[dv.at[pl.ds(j * W, W)]],
                    hv.at[pl.ds(j * W, W)], gs)

        def wait_gathers(b):
            _, _, hu, hv, gs = bufs[b]
            pltpu.make_async_copy(h2.at[pl.ds(0, C)], hu, gs).wait()
            pltpu.make_async_copy(h2.at[pl.ds(0, C)], hv, gs).wait()

        def compute(k, b):
            _, _, hu, hv, _ = bufs[b]

            def dot_body(t, carry2):
                rows16 = t * 16 + iot
                s = jnp.zeros((16,), jnp.float32)
                for j in range(8):
                    cj = jnp.full((16,), j, jnp.int32)
                    s = s + (plsc.load_gather(hu, [rows16, cj])
                             * plsc.load_gather(hv, [rows16, cj]))
                sc_v[pl.ds(t * 16, 16)] = s
                return carry2

            lax.fori_loop(0, C // 16, dot_body, 0)
            pltpu.sync_copy(sc_v, out.at[pl.ds(ebase + k * C, C)])

        chunk_fire(0, 0)

        def body(ii, carry):
            a = 2 * ii
            wait_gathers(0)
            chunk_fire(a + 1, 1)
            compute(a, 0)
            wait_gathers(1)

            @pl.when(ii + 1 < ITERS // 2)
            def _():
                chunk_fire(a + 2, 0)

            compute(a + 1, 1)
            return carry

        lax.fori_loop(0, ITERS // 2, body, 0)

    return _score_k


BN = 2000


def _combine_body(acc_ref, y1_ref, deg_ref):
    a = acc_ref[0] + acc_ref[1]
    deg = a[:, 4:5]
    y1_ref[...] = a[:, 0:4] / jnp.maximum(deg, 1.0)
    deg_ref[...] = deg


_combine = pl.pallas_call(
    _combine_body,
    grid=(N // BN,),
    in_specs=[pl.BlockSpec((2, BN, 8), lambda i: (0, i, 0))],
    out_specs=[
        pl.BlockSpec((BN, 4), lambda i: (i, 0)),
        pl.BlockSpec((BN, 1), lambda i: (i, 0)),
    ],
    out_shape=[
        jax.ShapeDtypeStruct((N, 4), jnp.float32),
        jax.ShapeDtypeStruct((N, 1), jnp.float32),
    ],
)


def _dense_body(x_ref, y1_ref, deg_ref, acc2_ref, w1s, w1n, w2s, w2n, b1, b2,
                h2_ref):
    f32 = jnp.float32
    c0 = jnp.dot(w1s[...], w2s[...], preferred_element_type=f32)
    c1 = (jnp.dot(w1n[...], w2s[...], preferred_element_type=f32)
          + jnp.dot(w1s[...], w2n[...], preferred_element_type=f32))
    c2 = jnp.dot(w1n[...], w2n[...], preferred_element_type=f32)
    c3 = jnp.dot(b1[...], w2s[...], preferred_element_type=f32) + b2[...]
    c4 = jnp.dot(b1[...], w2n[...], preferred_element_type=f32)
    deg = deg_ref[...]
    y2 = (acc2_ref[0] + acc2_ref[1])[:, 0:4] / jnp.maximum(deg, 1.0)
    m = (deg > 0.0).astype(f32)
    h2_ref[...] = (jnp.dot(x_ref[...], c0, preferred_element_type=f32)
                   + jnp.dot(y1_ref[...], c1, preferred_element_type=f32)
                   + jnp.dot(y2, c2, preferred_element_type=f32)
                   + c3 + m * c4)


_dense = pl.pallas_call(
    _dense_body,
    grid=(N // BN,),
    in_specs=[
        pl.BlockSpec((BN, 4), lambda i: (i, 0)),
        pl.BlockSpec((BN, 4), lambda i: (i, 0)),
        pl.BlockSpec((BN, 1), lambda i: (i, 0)),
        pl.BlockSpec((2, BN, 8), lambda i: (0, i, 0)),
        pl.BlockSpec((4, 16), lambda i: (0, 0)),
        pl.BlockSpec((4, 16), lambda i: (0, 0)),
        pl.BlockSpec((16, 8), lambda i: (0, 0)),
        pl.BlockSpec((16, 8), lambda i: (0, 0)),
        pl.BlockSpec((1, 16), lambda i: (0, 0)),
        pl.BlockSpec((1, 8), lambda i: (0, 0)),
    ],
    out_specs=pl.BlockSpec((BN, 8), lambda i: (i, 0)),
    out_shape=jax.ShapeDtypeStruct((N, 8), jnp.float32),
)


def kernel(x, edge_index, neg_edge_index, W1_self, W1_neigh, b1, W2_self,
           W2_neigh, b2):
    del neg_edge_index
    ei = edge_index.astype(jnp.int32)
    xp = jnp.concatenate(
        [x, jnp.ones((N, 1), jnp.float32), jnp.zeros((N, 3), jnp.float32)],
        axis=1)
    z8 = jnp.zeros((RPT, 8), jnp.float32)

    acc1 = _make_scatter(8)(xp, ei, z8)
    y1, deg = _combine(acc1)
    y1p = jnp.concatenate([y1, jnp.zeros((N, 4), jnp.float32)], axis=1)
    acc2 = _make_scatter(8)(y1p, ei, z8)
    h2 = _dense(x, y1, deg, acc2, W1_self, W1_neigh, W2_self, W2_neigh,
                b1.reshape(1, 16), b2.reshape(1, 8))
    score = _make_score()(h2, ei)
    return score.reshape(E, 1)


# speedup vs baseline: 62.2839x; 1.1400x over previous
import functools

import jax
import jax.numpy as jnp
from jax import lax
from jax.experimental import pallas as pl
from jax.experimental.pallas import tpu as pltpu
from jax.experimental.pallas import tpu_sc as plsc

N = 100000
E = 3200000
NC, NS = 2, 16
NW = NC * NS
W = 80
EPW = E // NW

NTRS = 50
CS = NTRS * W
ITERS_S = EPW // CS

NTR = 25
C = NTR * W
ITERS = EPW // C

RPT = 6272
N_PAD = NS * RPT
RW = N_PAD // NW
RCH = RW // 2


@functools.lru_cache(maxsize=None)
def _mesh():
    return plsc.VectorSubcoreMesh(
        core_axis_name="c", subcore_axis_name="s", num_cores=NC,
        num_subcores=NS)


def _sc_params():
    return pltpu.CompilerParams(
        use_tc_tiling_on_sc=False, needs_layout_passes=False)


@functools.lru_cache(maxsize=None)
def _make_scatter():
    @functools.partial(
        pl.kernel,
        out_type=jax.ShapeDtypeStruct((NC, N_PAD, 8), jnp.float32),
        mesh=_mesh(),
        compiler_params=_sc_params(),
        scratch_types=[
            pltpu.VMEM((CS,), jnp.int32),
            pltpu.VMEM((CS,), jnp.int32),
            pltpu.VMEM((CS, 8), jnp.float32),
            pltpu.VMEM_SHARED((N_PAD, 8), jnp.float32),
            pltpu.SemaphoreType.DMA,
            pltpu.SemaphoreType.DMA,
        ],
    )
    def sk(table, ei, zrows, out, src_v, dst_v, rows_v, acc, gsem, ssem):
        cid = lax.axis_index("c")
        sid = lax.axis_index("s")
        wid = sid * NC + cid
        r0 = sid * RPT
        pltpu.sync_copy(zrows, acc.at[pl.ds(r0, RPT)])
        plsc.subcore_barrier()
        ebase = wid * EPW

        def body(i, carry):
            off = ebase + i * CS
            pltpu.sync_copy(ei.at[0, pl.ds(off, CS)], src_v)
            pltpu.sync_copy(ei.at[1, pl.ds(off, CS)], dst_v)
            g = [
                pltpu.async_copy(
                    table.at[src_v.at[pl.ds(j * W, W)]],
                    rows_v.at[pl.ds(j * W, W)], gsem)
                for j in range(NTRS)
            ]
            for d in g:
                d.wait()
            s = [
                pltpu.async_copy(
                    rows_v.at[pl.ds(j * W, W)],
                    acc.at[dst_v.at[pl.ds(j * W, W)]], ssem, add=True)
                for j in range(NTRS)
            ]
            for d in s:
                d.wait()
            return carry

        lax.fori_loop(0, ITERS_S, body, 0)
        plsc.subcore_barrier()
        pltpu.sync_copy(acc.at[pl.ds(r0, RPT)], out.at[cid, pl.ds(r0, RPT)])

    return sk


@functools.lru_cache(maxsize=None)
def _make_combine():
    @functools.partial(
        pl.kernel,
        out_type=jax.ShapeDtypeStruct((N_PAD, 8), jnp.float32),
        mesh=_mesh(),
        compiler_params=_sc_params(),
        scratch_types=[
            pltpu.VMEM((RCH, 8), jnp.float32),
            pltpu.VMEM((RCH, 8), jnp.float32),
            pltpu.VMEM((RCH, 8), jnp.float32),
        ],
    )
    def ck(acc1, out, a0_v, a1_v, yt_v):
        cid = lax.axis_index("c")
        sid = lax.axis_index("s")
        wid = sid * NC + cid
        iot = lax.iota(jnp.int32, 16)
        cols = jnp.bitwise_and(iot, 7)
        rows_base = lax.shift_right_logical(iot, 3)
        is03 = cols < 4
        is4 = cols == 4
        c4v = jnp.full((16,), 4, jnp.int32)
        zero16 = jnp.zeros((16,), jnp.float32)

        for h in range(RW // RCH):
            row0 = wid * RW + h * RCH
            pltpu.sync_copy(acc1.at[0, pl.ds(row0, RCH)], a0_v)
            pltpu.sync_copy(acc1.at[1, pl.ds(row0, RCH)], a1_v)

            def body(t, carry):
                r = t * 2 + rows_base
                va = (plsc.load_gather(a0_v, [r, cols])
                      + plsc.load_gather(a1_v, [r, cols]))
                vdeg = (plsc.load_gather(a0_v, [r, c4v])
                        + plsc.load_gather(a1_v, [r, c4v]))
                y = va / jnp.maximum(vdeg, 1.0)
                outv = jnp.where(is03, y, jnp.where(is4, vdeg, zero16))
                plsc.store_scatter(yt_v, [r, cols], outv)
                return carry

            lax.fori_loop(0, RCH // 2, body, 0)
            pltpu.sync_copy(yt_v, out.at[pl.ds(row0, RCH)])

    return ck


@functools.lru_cache(maxsize=None)
def _make_dense():
    @functools.partial(
        pl.kernel,
        out_type=jax.ShapeDtypeStruct((N_PAD, 8), jnp.float32),
        mesh=_mesh(),
        compiler_params=_sc_params(),
        scratch_types=[
            pltpu.VMEM((RCH, 8), jnp.float32),
            pltpu.VMEM((RCH, 8), jnp.float32),
            pltpu.VMEM((RCH, 8), jnp.float32),
            pltpu.VMEM((RCH, 8), jnp.float32),
            pltpu.VMEM((RCH, 8), jnp.float32),
            pltpu.VMEM((12, 8), jnp.float32),
            pltpu.VMEM((2, 8), jnp.float32),
        ],
    )
    def dk(xp, y1p, acc2, cm, cb, out, x_v, y_v, a0_v, a1_v, h_v, cm_v, cb_v):
        cid = lax.axis_index("c")
        sid = lax.axis_index("s")
        wid = sid * NC + cid
        pltpu.sync_copy(cm, cm_v)
        pltpu.sync_copy(cb, cb_v)
        iot = lax.iota(jnp.int32, 16)
        cols = jnp.bitwise_and(iot, 7)
        rows_base = lax.shift_right_logical(iot, 3)
        c4v = jnp.full((16,), 4, jnp.int32)
        zero16 = jnp.zeros((16,), jnp.float32)
        c0k = [plsc.load_gather(cm_v, [jnp.full((16,), k, jnp.int32), cols])
               for k in range(4)]
        c1k = [plsc.load_gather(cm_v, [jnp.full((16,), 4 + k, jnp.int32), cols])
               for k in range(4)]
        c2k = [plsc.load_gather(cm_v, [jnp.full((16,), 8 + k, jnp.int32), cols])
               for k in range(4)]
        c3v = plsc.load_gather(cb_v, [jnp.full((16,), 0, jnp.int32), cols])
        c4b = plsc.load_gather(cb_v, [jnp.full((16,), 1, jnp.int32), cols])

        for h in range(RW // RCH):
            row0 = wid * RW + h * RCH
            pltpu.sync_copy(xp.at[pl.ds(row0, RCH)], x_v)
            pltpu.sync_copy(y1p.at[pl.ds(row0, RCH)], y_v)
            pltpu.sync_copy(acc2.at[0, pl.ds(row0, RCH)], a0_v)
            pltpu.sync_copy(acc2.at[1, pl.ds(row0, RCH)], a1_v)

            def body(t, carry):
                r = t * 2 + rows_base
                vdeg = plsc.load_gather(y_v, [r, c4v])
                rdeg = 1.0 / jnp.maximum(vdeg, 1.0)
                macc = c3v + jnp.where(vdeg > 0.0, c4b, zero16)
                for k in range(4):
                    kc = jnp.full((16,), k, jnp.int32)
                    macc = macc + plsc.load_gather(x_v, [r, kc]) * c0k[k]
                    macc = macc + plsc.load_gather(y_v, [r, kc]) * c1k[k]
                    a2v = (plsc.load_gather(a0_v, [r, kc])
                           + plsc.load_gather(a1_v, [r, kc]))
                    macc = macc + a2v * rdeg * c2k[k]
                plsc.store_scatter(h_v, [r, cols], macc)
                return carry

            lax.fori_loop(0, RCH // 2, body, 0)
            pltpu.sync_copy(h_v, out.at[pl.ds(row0, RCH)])

    return dk


@functools.lru_cache(maxsize=None)
def _make_score():
    @functools.partial(
        pl.kernel,
        out_type=jax.ShapeDtypeStruct((E,), jnp.float32),
        mesh=_mesh(),
        compiler_params=_sc_params(),
        scratch_types=[
            pltpu.VMEM((C,), jnp.int32),
            pltpu.VMEM((C,), jnp.int32),
            pltpu.VMEM((C,), jnp.int32),
            pltpu.VMEM((C,), jnp.int32),
            pltpu.VMEM((C, 8), jnp.float32),
            pltpu.VMEM((C, 8), jnp.float32),
            pltpu.VMEM((C, 8), jnp.float32),
            pltpu.VMEM((C, 8), jnp.float32),
            pltpu.VMEM((C,), jnp.float32),
            pltpu.SemaphoreType.DMA,
            pltpu.SemaphoreType.DMA,
        ],
    )
    def _score_k(h2, ei, out, sv0, dv0, sv1, dv1, hu0, hv0, hu1, hv1, sc_v,
                 g0, g1):
        bufs = ((sv0, dv0, hu0, hv0, g0), (sv1, dv1, hu1, hv1, g1))
        cid = lax.axis_index("c")
        sid = lax.axis_index("s")
        wid = sid * NC + cid
        ebase = wid * EPW
        iot = lax.iota(jnp.int32, 16)

        def chunk_fire(k, b):
            sv, dv, hu, hv, gs = bufs[b]
            off = ebase + k * C
            pltpu.sync_copy(ei.at[0, pl.ds(off, C)], sv)
            pltpu.sync_copy(ei.at[1, pl.ds(off, C)], dv)
            for j in range(NTR):
                pltpu.async_copy(
                    h2.at[sv.at[pl.ds(j * W, W)]],
                    hu.at[pl.ds(j * W, W)], gs)
                pltpu.async_copy(
                    h2.at[dv.at[pl.ds(j * W, W)]],
                    hv.at[pl.ds(j * W, W)], gs)

        def wait_gathers(b):
            _, _, hu, hv, gs = bufs[b]
            pltpu.make_async_copy(h2.at[pl.ds(0, C)], hu, gs).wait()
            pltpu.make_async_copy(h2.at[pl.ds(0, C)], hv, gs).wait()

        def compute(k, b):
            _, _, hu, hv, _ = bufs[b]

            def dot_body(t, carry2):
                rows16 = t * 16 + iot
                s = jnp.zeros((16,), jnp.float32)
                for j in range(8):
                    cj = jnp.full((16,), j, jnp.int32)
                    s = s + (plsc.load_gather(hu, [rows16, cj])
                             * plsc.load_gather(hv, [rows16, cj]))
                sc_v[pl.ds(t * 16, 16)] = s
                return carry2

            lax.fori_loop(0, C // 16, dot_body, 0)
            pltpu.sync_copy(sc_v, out.at[pl.ds(ebase + k * C, C)])

        chunk_fire(0, 0)

        def body(ii, carry):
            a = 2 * ii
            wait_gathers(0)
            chunk_fire(a + 1, 1)
            compute(a, 0)
            wait_gathers(1)

            @pl.when(ii + 1 < ITERS // 2)
            def _():
                chunk_fire(a + 2, 0)

            compute(a + 1, 1)
            return carry

        lax.fori_loop(0, ITERS // 2, body, 0)

    return _score_k


def _coeff_body(w1s, w1n, w2s, w2n, b1, b2, cm_ref, cb_ref):
    f32 = jnp.float32
    cm_ref[0:4, :] = jnp.dot(w1s[...], w2s[...], preferred_element_type=f32)
    cm_ref[4:8, :] = (
        jnp.dot(w1n[...], w2s[...], preferred_element_type=f32)
        + jnp.dot(w1s[...], w2n[...], preferred_element_type=f32))
    cm_ref[8:12, :] = jnp.dot(w1n[...], w2n[...], preferred_element_type=f32)
    cb_ref[0:1, :] = jnp.dot(b1[...], w2s[...], preferred_element_type=f32) + b2[...]
    cb_ref[1:2, :] = jnp.dot(b1[...], w2n[...], preferred_element_type=f32)


_coeff = pl.pallas_call(
    _coeff_body,
    out_shape=[
        jax.ShapeDtypeStruct((12, 8), jnp.float32),
        jax.ShapeDtypeStruct((2, 8), jnp.float32),
    ],
)


def kernel(x, edge_index, neg_edge_index, W1_self, W1_neigh, b1, W2_self,
           W2_neigh, b2):
    del neg_edge_index
    ei = edge_index.astype(jnp.int32)
    xp = jnp.zeros((N_PAD, 8), jnp.float32)
    xp = xp.at[:N, 0:4].set(x).at[:N, 4].set(1.0)
    z8 = jnp.zeros((RPT, 8), jnp.float32)

    scatter = _make_scatter()
    acc1 = scatter(xp, ei, z8)
    y1p = _make_combine()(acc1)
    acc2 = scatter(y1p, ei, z8)
    cm, cb = _coeff(W1_self, W1_neigh, W2_self, W2_neigh,
                    b1.reshape(1, 16), b2.reshape(1, 8))
    h2 = _make_dense()(xp, y1p, acc2, cm, cb)
    score = _make_score()(h2, ei)
    return score.reshape(E, 1)


# speedup vs baseline: 72.3552x; 1.1617x over previous
import functools

import jax
import jax.numpy as jnp
from jax import lax
from jax.experimental import pallas as pl
from jax.experimental.pallas import tpu as pltpu
from jax.experimental.pallas import tpu_sc as plsc

N = 100000
E = 3200000
NC, NS = 2, 16
NW = NC * NS
W = 80
EPW = E // NW

NTRS = 50
CS = NTRS * W
ITERS_S = EPW // CS

NTR = 25
C = NTR * W
ITERS = EPW // C

RPT = 6272
N_PAD = NS * RPT
RW = N_PAD // NW
RCH = RW // 2


@functools.lru_cache(maxsize=None)
def _mesh():
    return plsc.VectorSubcoreMesh(
        core_axis_name="c", subcore_axis_name="s", num_cores=NC,
        num_subcores=NS)


def _sc_params():
    return pltpu.CompilerParams(
        use_tc_tiling_on_sc=False, needs_layout_passes=False)


@functools.lru_cache(maxsize=None)
def _make_scatter():
    @functools.partial(
        pl.kernel,
        out_type=jax.ShapeDtypeStruct((NC, N_PAD, 8), jnp.float32),
        mesh=_mesh(),
        compiler_params=_sc_params(),
        scratch_types=[
            pltpu.VMEM((CS,), jnp.int32),
            pltpu.VMEM((CS,), jnp.int32),
            pltpu.VMEM((CS, 8), jnp.float32),
            pltpu.VMEM_SHARED((N_PAD, 8), jnp.float32),
            pltpu.SemaphoreType.DMA,
            pltpu.SemaphoreType.DMA,
        ],
    )
    def sk(table, ei, out, src_v, dst_v, rows_v, acc, gsem, ssem):
        cid = lax.axis_index("c")
        sid = lax.axis_index("s")
        wid = sid * NC + cid
        r0 = sid * RPT
        iot = lax.iota(jnp.int32, 16)
        cols = jnp.bitwise_and(iot, 7)
        rows_base = lax.shift_right_logical(iot, 3)
        zero16 = jnp.zeros((16,), jnp.float32)

        def zbody(t, carry):
            plsc.store_scatter(rows_v, [t * 2 + rows_base, cols], zero16)
            return carry

        lax.fori_loop(0, CS // 2, zbody, 0)
        pltpu.sync_copy(rows_v, acc.at[pl.ds(r0, CS)])
        pltpu.sync_copy(rows_v.at[pl.ds(0, RPT - CS)],
                        acc.at[pl.ds(r0 + CS, RPT - CS)])
        plsc.subcore_barrier()
        ebase = wid * EPW

        def body(i, carry):
            off = ebase + i * CS
            pltpu.sync_copy(ei.at[0, pl.ds(off, CS)], src_v)
            pltpu.sync_copy(ei.at[1, pl.ds(off, CS)], dst_v)
            g = [
                pltpu.async_copy(
                    table.at[src_v.at[pl.ds(j * W, W)]],
                    rows_v.at[pl.ds(j * W, W)], gsem)
                for j in range(NTRS)
            ]
            for d in g:
                d.wait()
            s = [
                pltpu.async_copy(
                    rows_v.at[pl.ds(j * W, W)],
                    acc.at[dst_v.at[pl.ds(j * W, W)]], ssem, add=True)
                for j in range(NTRS)
            ]
            for d in s:
                d.wait()
            return carry

        lax.fori_loop(0, ITERS_S, body, 0)
        plsc.subcore_barrier()
        pltpu.sync_copy(acc.at[pl.ds(r0, RPT)], out.at[cid, pl.ds(r0, RPT)])

    return sk


NLAST = N - (NW - 1) * RW - RCH


@functools.lru_cache(maxsize=None)
def _make_pack():
    @functools.partial(
        pl.kernel,
        out_type=jax.ShapeDtypeStruct((N_PAD, 8), jnp.float32),
        mesh=_mesh(),
        compiler_params=_sc_params(),
        scratch_types=[
            pltpu.VMEM((RCH, 4), jnp.float32),
            pltpu.VMEM((RCH, 8), jnp.float32),
        ],
    )
    def pk(x, out, x_v, p_v):
        cid = lax.axis_index("c")
        sid = lax.axis_index("s")
        wid = sid * NC + cid
        iot = lax.iota(jnp.int32, 16)
        cols = jnp.bitwise_and(iot, 7)
        colsx = jnp.bitwise_and(cols, 3)
        rows_base = lax.shift_right_logical(iot, 3)
        is03 = cols < 4
        is4 = cols == 4
        one16 = jnp.full((16,), 1.0, jnp.float32)
        zero16 = jnp.zeros((16,), jnp.float32)

        for h in range(RW // RCH):
            row0 = wid * RW + h * RCH
            full = row0 + RCH <= N

            @pl.when(full)
            def _():
                pltpu.sync_copy(x.at[pl.ds(row0, RCH)], x_v)

            @pl.when(jnp.logical_not(full))
            def _():
                pltpu.sync_copy(x.at[pl.ds(row0, NLAST)],
                                x_v.at[pl.ds(0, NLAST)])

            def body(t, carry):
                r = t * 2 + rows_base
                valid = (row0 + r) < N
                xv = plsc.load_gather(x_v, [r, colsx])
                outv = jnp.where(
                    jnp.logical_and(valid, is03), xv,
                    jnp.where(jnp.logical_and(valid, is4), one16, zero16))
                plsc.store_scatter(p_v, [r, cols], outv)
                return carry

            lax.fori_loop(0, RCH // 2, body, 0)
            pltpu.sync_copy(p_v, out.at[pl.ds(row0, RCH)])

    return pk


@functools.lru_cache(maxsize=None)
def _make_combine():
    @functools.partial(
        pl.kernel,
        out_type=jax.ShapeDtypeStruct((N_PAD, 8), jnp.float32),
        mesh=_mesh(),
        compiler_params=_sc_params(),
        scratch_types=[
            pltpu.VMEM((RCH, 8), jnp.float32),
            pltpu.VMEM((RCH, 8), jnp.float32),
            pltpu.VMEM((RCH, 8), jnp.float32),
        ],
    )
    def ck(acc1, out, a0_v, a1_v, yt_v):
        cid = lax.axis_index("c")
        sid = lax.axis_index("s")
        wid = sid * NC + cid
        iot = lax.iota(jnp.int32, 16)
        cols = jnp.bitwise_and(iot, 7)
        rows_base = lax.shift_right_logical(iot, 3)
        is03 = cols < 4
        is4 = cols == 4
        c4v = jnp.full((16,), 4, jnp.int32)
        zero16 = jnp.zeros((16,), jnp.float32)

        for h in range(RW // RCH):
            row0 = wid * RW + h * RCH
            pltpu.sync_copy(acc1.at[0, pl.ds(row0, RCH)], a0_v)
            pltpu.sync_copy(acc1.at[1, pl.ds(row0, RCH)], a1_v)

            def body(t, carry):
                r = t * 2 + rows_base
                va = (plsc.load_gather(a0_v, [r, cols])
                      + plsc.load_gather(a1_v, [r, cols]))
                vdeg = (plsc.load_gather(a0_v, [r, c4v])
                        + plsc.load_gather(a1_v, [r, c4v]))
                y = va / jnp.maximum(vdeg, 1.0)
                outv = jnp.where(is03, y, jnp.where(is4, vdeg, zero16))
                plsc.store_scatter(yt_v, [r, cols], outv)
                return carry

            lax.fori_loop(0, RCH // 2, body, 0)
            pltpu.sync_copy(yt_v, out.at[pl.ds(row0, RCH)])

    return ck


@functools.lru_cache(maxsize=None)
def _make_dense():
    @functools.partial(
        pl.kernel,
        out_type=jax.ShapeDtypeStruct((N_PAD, 8), jnp.float32),
        mesh=_mesh(),
        compiler_params=_sc_params(),
        scratch_types=[
            pltpu.VMEM((RCH, 8), jnp.float32),
            pltpu.VMEM((RCH, 8), jnp.float32),
            pltpu.VMEM((RCH, 8), jnp.float32),
            pltpu.VMEM((RCH, 8), jnp.float32),
            pltpu.VMEM((RCH, 8), jnp.float32),
            pltpu.VMEM((12, 8), jnp.float32),
            pltpu.VMEM((2, 8), jnp.float32),
        ],
    )
    def dk(xp, y1p, acc2, cm, cb, out, x_v, y_v, a0_v, a1_v, h_v, cm_v, cb_v):
        cid = lax.axis_index("c")
        sid = lax.axis_index("s")
        wid = sid * NC + cid
        pltpu.sync_copy(cm, cm_v)
        pltpu.sync_copy(cb, cb_v)
        iot = lax.iota(jnp.int32, 16)
        cols = jnp.bitwise_and(iot, 7)
        rows_base = lax.shift_right_logical(iot, 3)
        c4v = jnp.full((16,), 4, jnp.int32)
        zero16 = jnp.zeros((16,), jnp.float32)
        c0k = [plsc.load_gather(cm_v, [jnp.full((16,), k, jnp.int32), cols])
               for k in range(4)]
        c1k = [plsc.load_gather(cm_v, [jnp.full((16,), 4 + k, jnp.int32), cols])
               for k in range(4)]
        c2k = [plsc.load_gather(cm_v, [jnp.full((16,), 8 + k, jnp.int32), cols])
               for k in range(4)]
        c3v = plsc.load_gather(cb_v, [jnp.full((16,), 0, jnp.int32), cols])
        c4b = plsc.load_gather(cb_v, [jnp.full((16,), 1, jnp.int32), cols])

        for h in range(RW // RCH):
            row0 = wid * RW + h * RCH
            pltpu.sync_copy(xp.at[pl.ds(row0, RCH)], x_v)
            pltpu.sync_copy(y1p.at[pl.ds(row0, RCH)], y_v)
            pltpu.sync_copy(acc2.at[0, pl.ds(row0, RCH)], a0_v)
            pltpu.sync_copy(acc2.at[1, pl.ds(row0, RCH)], a1_v)

            def body(t, carry):
                r = t * 2 + rows_base
                vdeg = plsc.load_gather(y_v, [r, c4v])
                rdeg = 1.0 / jnp.maximum(vdeg, 1.0)
                macc = c3v + jnp.where(vdeg > 0.0, c4b, zero16)
                for k in range(4):
                    kc = jnp.full((16,), k, jnp.int32)
                    macc = macc + plsc.load_gather(x_v, [r, kc]) * c0k[k]
                    macc = macc + plsc.load_gather(y_v, [r, kc]) * c1k[k]
                    a2v = (plsc.load_gather(a0_v, [r, kc])
                           + plsc.load_gather(a1_v, [r, kc]))
                    macc = macc + a2v * rdeg * c2k[k]
                plsc.store_scatter(h_v, [r, cols], macc)
                return carry

            lax.fori_loop(0, RCH // 2, body, 0)
            pltpu.sync_copy(h_v, out.at[pl.ds(row0, RCH)])

    return dk


@functools.lru_cache(maxsize=None)
def _make_score():
    @functools.partial(
        pl.kernel,
        out_type=jax.ShapeDtypeStruct((E,), jnp.float32),
        mesh=_mesh(),
        compiler_params=_sc_params(),
        scratch_types=[
            pltpu.VMEM((C,), jnp.int32),
            pltpu.VMEM((C,), jnp.int32),
            pltpu.VMEM((C,), jnp.int32),
            pltpu.VMEM((C,), jnp.int32),
            pltpu.VMEM((C, 8), jnp.float32),
            pltpu.VMEM((C, 8), jnp.float32),
            pltpu.VMEM((C, 8), jnp.float32),
            pltpu.VMEM((C, 8), jnp.float32),
            pltpu.VMEM((C,), jnp.float32),
            pltpu.SemaphoreType.DMA,
            pltpu.SemaphoreType.DMA,
        ],
    )
    def _score_k(h2, ei, out, sv0, dv0, sv1, dv1, hu0, hv0, hu1, hv1, sc_v,
                 g0, g1):
        bufs = ((sv0, dv0, hu0, hv0, g0), (sv1, dv1, hu1, hv1, g1))
        cid = lax.axis_index("c")
        sid = lax.axis_index("s")
        wid = sid * NC + cid
        ebase = wid * EPW
        iot = lax.iota(jnp.int32, 16)

        def chunk_fire(k, b):
            sv, dv, hu, hv, gs = bufs[b]
            off = ebase + k * C
            pltpu.sync_copy(ei.at[0, pl.ds(off, C)], sv)
            pltpu.sync_copy(ei.at[1, pl.ds(off, C)], dv)
            for j in range(NTR):
                pltpu.async_copy(
                    h2.at[sv.at[pl.ds(j * W, W)]],
                    hu.at[pl.ds(j * W, W)], gs)
                pltpu.async_copy(
                    h2.at[dv.at[pl.ds(j * W, W)]],
                    hv.at[pl.ds(j * W, W)], gs)

        def wait_gathers(b):
            _, _, hu, hv, gs = bufs[b]
            pltpu.make_async_copy(h2.at[pl.ds(0, C)], hu, gs).wait()
            pltpu.make_async_copy(h2.at[pl.ds(0, C)], hv, gs).wait()

        def compute(k, b):
            _, _, hu, hv, _ = bufs[b]

            def dot_body(t, carry2):
                rows16 = t * 16 + iot
                s = jnp.zeros((16,), jnp.float32)
                for j in range(8):
                    cj = jnp.full((16,), j, jnp.int32)
                    s = s + (plsc.load_gather(hu, [rows16, cj])
                             * plsc.load_gather(hv, [rows16, cj]))
                sc_v[pl.ds(t * 16, 16)] = s
                return carry2

            lax.fori_loop(0, C // 16, dot_body, 0)
            pltpu.sync_copy(sc_v, out.at[pl.ds(ebase + k * C, C)])

        chunk_fire(0, 0)

        def body(ii, carry):
            a = 2 * ii
            wait_gathers(0)
            chunk_fire(a + 1, 1)
            compute(a, 0)
            wait_gathers(1)

            @pl.when(ii + 1 < ITERS // 2)
            def _():
                chunk_fire(a + 2, 0)

            compute(a + 1, 1)
            return carry

        lax.fori_loop(0, ITERS // 2, body, 0)

    return _score_k


def _coeff_body(w1s, w1n, w2s, w2n, b1, b2, cm_ref, cb_ref):
    f32 = jnp.float32
    cm_ref[0:4, :] = jnp.dot(w1s[...], w2s[...], preferred_element_type=f32)
    cm_ref[4:8, :] = (
        jnp.dot(w1n[...], w2s[...], preferred_element_type=f32)
        + jnp.dot(w1s[...], w2n[...], preferred_element_type=f32))
    cm_ref[8:12, :] = jnp.dot(w1n[...], w2n[...], preferred_element_type=f32)
    cb_ref[0:1, :] = jnp.dot(b1[...], w2s[...], preferred_element_type=f32) + b2[...]
    cb_ref[1:2, :] = jnp.dot(b1[...], w2n[...], preferred_element_type=f32)


_coeff = pl.pallas_call(
    _coeff_body,
    out_shape=[
        jax.ShapeDtypeStruct((12, 8), jnp.float32),
        jax.ShapeDtypeStruct((2, 8), jnp.float32),
    ],
)


def kernel(x, edge_index, neg_edge_index, W1_self, W1_neigh, b1, W2_self,
           W2_neigh, b2):
    del neg_edge_index
    ei = edge_index.astype(jnp.int32)
    xp = _make_pack()(x)

    scatter = _make_scatter()
    acc1 = scatter(xp, ei)
    y1p = _make_combine()(acc1)
    acc2 = scatter(y1p, ei)
    cm, cb = _coeff(W1_self, W1_neigh, W2_self, W2_neigh,
                    b1.reshape(1, 16), b2.reshape(1, 8))
    h2 = _make_dense()(xp, y1p, acc2, cm, cb)
    score = _make_score()(h2, ei)
    return score.reshape(E, 1)
